# baseline (speedup 1.0000x reference)
import jax
import jax.numpy as jnp
from jax import lax
from jax.experimental import pallas as pl
from jax.experimental.pallas import tpu as pltpu
from jax.experimental.pallas import tpu_sc as plsc

RMAX = 5.0
AVG = 16.0
LMAP = (0, 1, 1, 1)
S3 = 1.7320508075688772


def _sc_gather_rows(table, idx, bt, tc_tiling=True):
    v, d = table.shape
    b = idx.shape[0]
    nw = 32
    per_w = b // nw
    iters = per_w // bt
    tail = per_w - iters * bt
    assert per_w * nw == b and bt % 8 == 0 and bt <= 128 and tail % 8 == 0

    def body(table_hbm, idx_hbm, out_hbm, idx_v, rows_v, sem_i, sem_g, sem_s):
        wid = lax.axis_index("s") * 2 + lax.axis_index("c")
        base = wid * per_w

        def load_idx(j, buf):
            pltpu.async_copy(idx_hbm.at[pl.ds(base + j * bt, bt)],
                             idx_v.at[buf], sem_i.at[buf])

        load_idx(0, 0)

        @pl.when(iters > 1)
        def _():
            load_idx(1, 1)

        def step(j, c):
            bc = j % 4
            pltpu.make_async_copy(idx_hbm.at[pl.ds(base, bt)],
                                  idx_v.at[bc], sem_i.at[bc]).wait()

            @pl.when(j >= 4)
            def _():
                pltpu.make_async_copy(rows_v.at[bc],
                                      out_hbm.at[pl.ds(base, bt)],
                                      sem_s.at[bc]).wait()

            pltpu.async_copy(table_hbm.at[idx_v.at[bc]], rows_v.at[bc],
                             sem_g.at[bc])

            @pl.when(j >= 2)
            def _():
                bp = (j - 2) % 4
                pltpu.make_async_copy(table_hbm.at[idx_v.at[bp]],
                                      rows_v.at[bp], sem_g.at[bp]).wait()
                pltpu.async_copy(rows_v.at[bp],
                                 out_hbm.at[pl.ds(base + (j - 2) * bt, bt)],
                                 sem_s.at[bp])

            @pl.when(j + 2 < iters)
            def _():
                load_idx(j + 2, (j + 2) % 4)

            return c

        lax.fori_loop(0, iters, step, 0)
        for jt in range(2):
            j = iters - 2 + jt
            if j < 0:
                continue
            bp = j % 4
            pltpu.make_async_copy(table_hbm.at[idx_v.at[bp]],
                                  rows_v.at[bp], sem_g.at[bp]).wait()
            pltpu.async_copy(rows_v.at[bp],
                             out_hbm.at[pl.ds(base + j * bt, bt)],
                             sem_s.at[bp])
        for jt in range(min(4, iters)):
            j = iters - min(4, iters) + jt
            bp = j % 4
            pltpu.make_async_copy(rows_v.at[bp],
                                  out_hbm.at[pl.ds(base, bt)],
                                  sem_s.at[bp]).wait()
        if tail:
            off = base + iters * bt
            pltpu.sync_copy(idx_hbm.at[pl.ds(off, tail)],
                            idx_v.at[0, pl.ds(0, tail)])
            pltpu.async_copy(table_hbm.at[idx_v.at[0, pl.ds(0, tail)]],
                             rows_v.at[0, pl.ds(0, tail)], sem_g.at[0]).wait()
            pltpu.sync_copy(rows_v.at[0, pl.ds(0, tail)],
                            out_hbm.at[pl.ds(off, tail)])

    fn = pl.kernel(
        body,
        out_type=jax.ShapeDtypeStruct((b, d), jnp.float32),
        mesh=plsc.VectorSubcoreMesh(core_axis_name="c", subcore_axis_name="s"),
        scratch_types=[
            pltpu.VMEM((4, bt), jnp.int32),
            pltpu.VMEM((4, bt, d), jnp.float32),
            pltpu.SemaphoreType.DMA((4,)),
            pltpu.SemaphoreType.DMA((4,)),
            pltpu.SemaphoreType.DMA((4,)),
        ],
        compiler_params=pltpu.CompilerParams(use_tc_tiling_on_sc=tc_tiling),
    )
    return fn(table, idx)


def _sc_scatter4(msg_flat, dst, zeros_hbm):
    n = zeros_hbm.shape[0]
    e = dst.shape[0]
    bt = 64
    per_t = e // 16
    iters = per_t // bt
    tail = per_t - iters * bt
    big = 640
    last = n - 15 * big
    assert tail % 8 == 0 and last > 0 and big % 8 == 0 and last % 8 == 0

    def body(msg_hbm, dst_hbm, zer_hbm, out_hbm, rows_v, idx_v, rows_t, idx_t,
             accum_sh, sem_l, sem_m, sem_sc):
        cid = lax.axis_index("c")
        sid = lax.axis_index("s")
        ebase = sid * per_t
        nbase = sid * big

        def stripe_copy(src_at, dst_at):
            @pl.when(sid < 15)
            def _():
                pltpu.sync_copy(src_at(nbase, big), dst_at(nbase, big))

            @pl.when(sid == 15)
            def _():
                pltpu.sync_copy(src_at(15 * big, last), dst_at(15 * big, last))

        for p in range(2):
            s_val = cid * 2 + p
            stripe_copy(lambda o, w: zer_hbm.at[pl.ds(o, w)],
                        lambda o, w: accum_sh.at[pl.ds(o, w)])
            plsc.subcore_barrier()

            def loads(j, buf):
                off = ebase + j * bt
                pltpu.async_copy(dst_hbm.at[pl.ds(off, bt)],
                                 idx_v.at[buf], sem_l.at[buf])
                pltpu.async_copy(msg_hbm.at[pl.ds(s_val * e + off, bt)],
                                 rows_v.at[buf], sem_m.at[buf])

            loads(0, 0)

            @pl.when(iters > 1)
            def _():
                loads(1, 1)

            def step(j, c):
                bc = j % 4
                pltpu.make_async_copy(dst_hbm.at[pl.ds(ebase, bt)],
                                      idx_v.at[bc], sem_l.at[bc]).wait()
                pltpu.make_async_copy(msg_hbm.at[pl.ds(ebase, bt)],
                                      rows_v.at[bc], sem_m.at[bc]).wait()
                pltpu.async_copy(rows_v.at[bc], accum_sh.at[idx_v.at[bc]],
                                 sem_sc.at[bc], add=True)

                @pl.when(j + 2 < iters)
                def _():
                    bn = (j + 2) % 4

                    @pl.when(j >= 2)
                    def _():
                        pltpu.make_async_copy(
                            rows_v.at[bn], accum_sh.at[idx_v.at[bn]],
                            sem_sc.at[bn]).wait()

                    loads(j + 2, bn)

                return c

            lax.fori_loop(0, iters, step, 0)
            for jt in range(min(4, iters)):
                j = iters - min(4, iters) + jt
                bp = j % 4
                pltpu.make_async_copy(rows_v.at[bp],
                                      accum_sh.at[idx_v.at[bp]],
                                      sem_sc.at[bp]).wait()
            if tail:
                off = ebase + iters * bt
                pltpu.sync_copy(dst_hbm.at[pl.ds(off, tail)], idx_t)
                pltpu.sync_copy(msg_hbm.at[pl.ds(s_val * e + off, tail)],
                                rows_t)
                pltpu.sync_copy(rows_t, accum_sh.at[idx_t], add=True)
            plsc.subcore_barrier()
            stripe_copy(lambda o, w: accum_sh.at[pl.ds(o, w)],
                        lambda o, w: out_hbm.at[pl.ds(s_val * n + o, w)])
            plsc.subcore_barrier()

    fn = pl.kernel(
        body,
        out_type=jax.ShapeDtypeStruct((4 * n, 128), jnp.float32),
        mesh=plsc.VectorSubcoreMesh(core_axis_name="c", subcore_axis_name="s"),
        scratch_types=[
            pltpu.VMEM((4, bt, 128), jnp.float32),
            pltpu.VMEM((4, bt), jnp.int32),
            pltpu.VMEM((max(tail, 8), 128), jnp.float32),
            pltpu.VMEM((max(tail, 8),), jnp.int32),
            pltpu.VMEM_SHARED((n, 128), jnp.float32),
            pltpu.SemaphoreType.DMA((4,)),
            pltpu.SemaphoreType.DMA((4,)),
            pltpu.SemaphoreType.DMA((4,)),
        ],
    )
    return fn(msg_flat, dst, zeros_hbm)


def _silu(x):
    return x * jax.nn.sigmoid(x)


def _mm(a, b):
    return jnp.dot(a, b, preferred_element_type=jnp.float32)


def _rmlp_block(ef, w1, b1, w2, b2, w3, b3, w4):
    h = _silu(_mm(ef, w1) + b1)
    h = _silu(_mm(h, w2) + b2)
    h = _silu(_mm(h, w3) + b3)
    return _mm(h, w4)


def _edge_geom(ps_ref, pd_ref, sh_ref):
    vec = pd_ref[...] - ps_ref[...] + sh_ref[...]
    len2 = jnp.sum(vec * vec, axis=1, keepdims=True) + 1e-18
    r = jnp.sqrt(len2)
    u = vec * (1.0 / r)

    rc = jnp.minimum(jnp.maximum(r, 1e-6), RMAX)
    narr = (lax.broadcasted_iota(jnp.int32, (1, 8), 1) + 1).astype(jnp.float32)
    rb = jnp.sqrt(2.0 / RMAX) * jnp.sin(rc * (jnp.pi / RMAX) * narr) / rc
    uu = jnp.clip(r / RMAX, 0.0, 1.0)
    u5 = uu * uu * uu * uu * uu
    env = 1.0 - 21.0 * u5 + 35.0 * u5 * uu - 15.0 * u5 * uu * uu
    env = jnp.where(r < RMAX, env, 0.0)
    ef = rb * env
    return u, ef


def _edge1_body(ps_ref, pd_ref, sh_ref, hs_ref,
                aw1, ab1, aw2, ab2, aw3, ab3, aw4,
                msg_ref):
    u, ef = _edge_geom(ps_ref, pd_ref, sh_ref)
    r0 = _rmlp_block(ef, aw1[...], ab1[...], aw2[...], ab2[...],
                     aw3[...], ab3[...], aw4[...])
    hs = hs_ref[...]
    for s in range(4):
        rs = r0[:, LMAP[s] * 128:(LMAP[s] + 1) * 128]
        if s == 0:
            msg_ref[s] = rs * hs
        else:
            msg_ref[s] = rs * ((S3 * u[:, s - 1:s]) * hs)


def _edge2_body(ps_ref, pd_ref, sh_ref, h2s_ref,
                bw1, bb1, bw2, bb2, bw3, bb3, bw4,
                msg_ref):
    u, ef = _edge_geom(ps_ref, pd_ref, sh_ref)
    r1 = _rmlp_block(ef, bw1[...], bb1[...], bw2[...], bb2[...],
                     bw3[...], bb3[...], bw4[...])
    h2s = h2s_ref[...]
    h0 = h2s[:, 0:128]
    for s in range(4):
        rs = r1[:, LMAP[s] * 128:(LMAP[s] + 1) * 128]
        if s == 0:
            ys_h0 = h0
        else:
            ys_h0 = (S3 * u[:, s - 1:s]) * h0
        msg_ref[s] = rs * (ys_h0 + h2s[:, s * 128:(s + 1) * 128])


def _node0_body(na_ref, wemb, wup0, h_ref):
    h_ref[...] = _mm(_mm(na_ref[...], wemb[...]), wup0[...])


def _graph_accum(eg_ref, batch_col, en_node):
    iot = lax.broadcasted_iota(jnp.int32, (batch_col.shape[0], 16), 1)
    oh = (batch_col == iot).astype(jnp.float32)
    part = lax.dot_general(oh, en_node, (((0,), (0,)), ((), ())),
                           preferred_element_type=jnp.float32)
    i = pl.program_id(0)

    @pl.when(i == 0)
    def _():
        eg_ref[...] = jnp.zeros_like(eg_ref)

    eg_ref[...] += part


def _node1_body(agg_ref, na_ref, bat_ref,
                wout_r, pw_r, pv_r, wsc_r, wmix_r, wmixv_r, wemb_r, wup1_r,
                ae_r, ro0_r, h2_ref, scal1_ref, eg_ref):
    wout, pw, pv, wsc, wmix, wmixv, wemb, wup1, ae, ro0 = (
        wout_r[...], pw_r[...], pv_r[...], wsc_r[...], wmix_r[...],
        wmixv_r[...], wemb_r[...], wup1_r[...], ae_r[...], ro0_r[...])
    na = na_ref[...]
    nf = _mm(na, wemb)
    out0 = _mm(agg_ref[0] * (1.0 / AVG), wout[0])
    outv = [_mm(agg_ref[s] * (1.0 / AVG), wout[1]) for s in range(1, 4)]
    w0 = _mm(na, pw[0])
    w1 = _mm(na, pw[1])
    w2 = _mm(na, pw[2])
    vsq = outv[0] * outv[0] + outv[1] * outv[1] + outv[2] * outv[2]
    scal = w0 * out0 + w1 * out0 * out0 + w2 * vsq
    sc = _mm(na, wsc) * nf
    scal = _mm(scal, wmix) + sc
    pv0 = _mm(na, pv[0])
    pv1 = _mm(na, pv[1])
    coef = pv0 + pv1 * out0
    vout = [_mm(coef * v, wmixv) for v in outv]

    scal1_ref[...] = scal
    h2_ref[:, 0:128] = _mm(scal, wup1[0])
    for s in range(1, 4):
        h2_ref[:, s * 128:(s + 1) * 128] = _mm(vout[s - 1], wup1[1])

    e01 = _mm(na, ae) + _mm(scal, ro0)
    _graph_accum(eg_ref, bat_ref[...], e01)


def _node2_body(agg_ref, na_ref, bat_ref, scal1_ref,
                wout_r, pw_r, wsc_r, wmix_r, row1_r, rob1, row2_r,
                eg_ref):
    wout, pw, wsc, wmix, row1, row2 = (
        wout_r[...], pw_r[...], wsc_r[...], wmix_r[...], row1_r[...],
        row2_r[...])
    na = na_ref[...]
    out0 = _mm(agg_ref[0] * (1.0 / AVG), wout[0])
    outv = [_mm(agg_ref[s] * (1.0 / AVG), wout[1]) for s in range(1, 4)]
    w0 = _mm(na, pw[0])
    w1 = _mm(na, pw[1])
    w2 = _mm(na, pw[2])
    vsq = outv[0] * outv[0] + outv[1] * outv[1] + outv[2] * outv[2]
    scal2 = w0 * out0 + w1 * out0 * out0 + w2 * vsq
    sc2 = _mm(na, wsc) * scal1_ref[...]
    scal2 = _mm(scal2, wmix) + sc2
    hr = _silu(_mm(scal2, row1) + rob1[...])
    en2 = _mm(hr, row2)
    _graph_accum(eg_ref, bat_ref[...], en2)


def _full(shape):
    nd = len(shape)
    return pl.BlockSpec(shape, lambda i: (0,) * nd)


def _edge_specs(b1, g, feat_w):
    return [
        pl.BlockSpec((b1, 16), lambda i: (i, 0)),
        pl.BlockSpec((b1, 16), lambda i: (i + g, 0)),
        pl.BlockSpec((b1, 16), lambda i: (i, 0)),
        pl.BlockSpec((b1, feat_w), lambda i: (i, 0)),
        _full((8, 64)), _full((1, 64)), _full((64, 64)), _full((1, 64)),
        _full((64, 64)), _full((1, 64)), _full((64, 256)),
    ]


def _run_edge1(posg, shifts16, h_src, lw, e, b1, interpret=False):
    g = e // b1
    return pl.pallas_call(
        _edge1_body, grid=(g,), in_specs=_edge_specs(b1, g, 128),
        out_specs=pl.BlockSpec((4, b1, 128), lambda i: (0, i, 0)),
        out_shape=jax.ShapeDtypeStruct((4, e, 128), jnp.float32),
        interpret=interpret,
    )(posg, posg, shifts16, h_src, *lw)


def _run_edge2(posg, shifts16, h2s, lw, e, b1, interpret=False):
    g = e // b1
    return pl.pallas_call(
        _edge2_body, grid=(g,), in_specs=_edge_specs(b1, g, 512),
        out_specs=pl.BlockSpec((4, b1, 128), lambda i: (0, i, 0)),
        out_shape=jax.ShapeDtypeStruct((4, e, 128), jnp.float32),
        interpret=interpret,
    )(posg, posg, shifts16, h2s, *lw)


def _run_node0(na, wemb, wup0, n, bn, interpret=False):
    return pl.pallas_call(
        _node0_body, grid=(n // bn,),
        in_specs=[pl.BlockSpec((bn, 10), lambda i: (i, 0)),
                  _full((10, 128)), _full((128, 128))],
        out_specs=pl.BlockSpec((bn, 128), lambda i: (i, 0)),
        out_shape=jax.ShapeDtypeStruct((n, 128), jnp.float32),
        interpret=interpret,
    )(na, wemb, wup0)


def _run_node1(agg0, na, bat2, wts, n, bn, interpret=False):
    g = n // bn
    specs = [
        pl.BlockSpec((4, bn, 128), lambda i: (0, i, 0)),
        pl.BlockSpec((bn, 10), lambda i: (i, 0)),
        pl.BlockSpec((bn, 1), lambda i: (i, 0)),
        _full((2, 128, 128)), _full((3, 10, 128)), _full((2, 10, 128)),
        _full((10, 128)), _full((128, 128)), _full((128, 128)),
        _full((10, 128)), _full((2, 128, 128)), _full((10, 1)),
        _full((128, 1)),
    ]
    out_specs = [
        pl.BlockSpec((bn, 512), lambda i: (i, 0)),
        pl.BlockSpec((bn, 128), lambda i: (i, 0)),
        pl.BlockSpec((16, 1), lambda i: (0, 0)),
    ]
    out_shape = [
        jax.ShapeDtypeStruct((n, 512), jnp.float32),
        jax.ShapeDtypeStruct((n, 128), jnp.float32),
        jax.ShapeDtypeStruct((16, 1), jnp.float32),
    ]
    return pl.pallas_call(
        _node1_body, grid=(g,), in_specs=specs, out_specs=out_specs,
        out_shape=out_shape, interpret=interpret,
    )(agg0, na, bat2, *wts)


def _run_node2(agg2, na, bat2, scal1, wts, n, bn, interpret=False):
    g = n // bn
    specs = [
        pl.BlockSpec((4, bn, 128), lambda i: (0, i, 0)),
        pl.BlockSpec((bn, 10), lambda i: (i, 0)),
        pl.BlockSpec((bn, 1), lambda i: (i, 0)),
        pl.BlockSpec((bn, 128), lambda i: (i, 0)),
        _full((2, 128, 128)), _full((3, 10, 128)), _full((10, 128)),
        _full((128, 128)), _full((128, 16)), _full((1, 16)), _full((16, 1)),
    ]
    return pl.pallas_call(
        _node2_body, grid=(g,), in_specs=specs,
        out_specs=pl.BlockSpec((16, 1), lambda i: (0, 0)),
        out_shape=jax.ShapeDtypeStruct((16, 1), jnp.float32),
        interpret=interpret,
    )(agg2, na, bat2, scal1, *wts)


def kernel(positions, node_attrs, shifts, params, edge_index, batch, ptr):
    n = positions.shape[0]
    e = edge_index.shape[1]
    ng = ptr.shape[0] - 1
    b1 = 1000
    bn = 1000

    p0 = params["layer0"]
    p1 = params["layer1"]
    l0w = (p0["rW1"], p0["rb1"].reshape(1, 64), p0["rW2"], p0["rb2"].reshape(1, 64),
           p0["rW3"], p0["rb3"].reshape(1, 64), p0["rW4"])
    l1w = (p1["rW1"], p1["rb1"].reshape(1, 64), p1["rW2"], p1["rb2"].reshape(1, 64),
           p1["rW3"], p1["rb3"].reshape(1, 64), p1["rW4"])

    pos16 = jnp.pad(positions, ((0, 0), (0, 13)))
    sh16 = jnp.pad(shifts, ((0, 0), (0, 13)))
    eidx = edge_index.reshape(2 * e)
    src = edge_index[0]
    dst = edge_index[1]
    bat2 = batch.reshape(n, 1)
    zeros_n = jnp.zeros((n, 128), jnp.float32)

    h = _run_node0(node_attrs, params["W_embed"], p0["W_up"], n, bn)

    posg = _sc_gather_rows(pos16, eidx, 128, tc_tiling=False)
    h_src = _sc_gather_rows(h, src, 128)

    msg0 = _run_edge1(posg, sh16, h_src, l0w, e, b1)
    agg0 = _sc_scatter4(msg0.reshape(4 * e, 128), dst, zeros_n).reshape(4, n, 128)

    wts1 = (p0["W_out"], p0["pw"], p0["pv"], p0["W_sc"], p0["W_mix"],
            p0["W_mixv"], params["W_embed"], p1["W_up"],
            params["atomic_energies"].reshape(10, 1),
            params["readout0"].reshape(128, 1))
    h2, scal1, e01g = _run_node1(agg0, node_attrs, bat2, wts1, n, bn)

    h2s = _sc_gather_rows(h2, src, 56)
    msg2 = _run_edge2(posg, sh16, h2s, l1w, e, b1)
    agg2 = _sc_scatter4(msg2.reshape(4 * e, 128), dst, zeros_n).reshape(4, n, 128)

    wts2 = (p1["W_out"], p1["pw"], p1["W_sc"], p1["W_mix"],
            params["ro1_W1"], params["ro1_b1"].reshape(1, 16),
            params["ro1_W2"])
    e2g = _run_node2(agg2, node_attrs, bat2, scal1, wts2, n, bn)

    return (e01g + e2g).reshape(ng)


# speedup vs baseline: 1.1602x; 1.1602x over previous
import jax
import jax.numpy as jnp
from jax import lax
from jax.experimental import pallas as pl
from jax.experimental.pallas import tpu as pltpu
from jax.experimental.pallas import tpu_sc as plsc

RMAX = 5.0
AVG = 16.0
LMAP = (0, 1, 1, 1)
S3 = 1.7320508075688772


def _sc_gather_rows(table, idx, bt, tc_tiling=True):
    v, d = table.shape
    b = idx.shape[0]
    nw = 32
    per_w = b // nw
    iters = per_w // bt
    tail = per_w - iters * bt
    assert per_w * nw == b and bt % 8 == 0 and bt <= 128 and tail % 8 == 0

    def body(table_hbm, idx_hbm, out_hbm, idx_v, rows_v, sem_i, sem_g, sem_s):
        wid = lax.axis_index("s") * 2 + lax.axis_index("c")
        base = wid * per_w

        def load_idx(j, buf):
            pltpu.async_copy(idx_hbm.at[pl.ds(base + j * bt, bt)],
                             idx_v.at[buf], sem_i.at[buf])

        load_idx(0, 0)

        @pl.when(iters > 1)
        def _():
            load_idx(1, 1)

        def step(j, c):
            bc = j % 4
            pltpu.make_async_copy(idx_hbm.at[pl.ds(base, bt)],
                                  idx_v.at[bc], sem_i.at[bc]).wait()

            @pl.when(j >= 4)
            def _():
                pltpu.make_async_copy(rows_v.at[bc],
                                      out_hbm.at[pl.ds(base, bt)],
                                      sem_s.at[bc]).wait()

            pltpu.async_copy(table_hbm.at[idx_v.at[bc]], rows_v.at[bc],
                             sem_g.at[bc])

            @pl.when(j >= 2)
            def _():
                bp = (j - 2) % 4
                pltpu.make_async_copy(table_hbm.at[idx_v.at[bp]],
                                      rows_v.at[bp], sem_g.at[bp]).wait()
                pltpu.async_copy(rows_v.at[bp],
                                 out_hbm.at[pl.ds(base + (j - 2) * bt, bt)],
                                 sem_s.at[bp])

            @pl.when(j + 2 < iters)
            def _():
                load_idx(j + 2, (j + 2) % 4)

            return c

        lax.fori_loop(0, iters, step, 0)
        for jt in range(2):
            j = iters - 2 + jt
            if j < 0:
                continue
            bp = j % 4
            pltpu.make_async_copy(table_hbm.at[idx_v.at[bp]],
                                  rows_v.at[bp], sem_g.at[bp]).wait()
            pltpu.async_copy(rows_v.at[bp],
                             out_hbm.at[pl.ds(base + j * bt, bt)],
                             sem_s.at[bp])
        for jt in range(min(4, iters)):
            j = iters - min(4, iters) + jt
            bp = j % 4
            pltpu.make_async_copy(rows_v.at[bp],
                                  out_hbm.at[pl.ds(base, bt)],
                                  sem_s.at[bp]).wait()
        if tail:
            off = base + iters * bt
            pltpu.sync_copy(idx_hbm.at[pl.ds(off, tail)],
                            idx_v.at[0, pl.ds(0, tail)])
            pltpu.async_copy(table_hbm.at[idx_v.at[0, pl.ds(0, tail)]],
                             rows_v.at[0, pl.ds(0, tail)], sem_g.at[0]).wait()
            pltpu.sync_copy(rows_v.at[0, pl.ds(0, tail)],
                            out_hbm.at[pl.ds(off, tail)])

    fn = pl.kernel(
        body,
        out_type=jax.ShapeDtypeStruct((b, d), jnp.float32),
        mesh=plsc.VectorSubcoreMesh(core_axis_name="c", subcore_axis_name="s"),
        scratch_types=[
            pltpu.VMEM((4, bt), jnp.int32),
            pltpu.VMEM((4, bt, d), jnp.float32),
            pltpu.SemaphoreType.DMA((4,)),
            pltpu.SemaphoreType.DMA((4,)),
            pltpu.SemaphoreType.DMA((4,)),
        ],
        compiler_params=pltpu.CompilerParams(use_tc_tiling_on_sc=tc_tiling),
    )
    return fn(table, idx)


def _sc_scatter4(msg_flat, dst, zeros_hbm):
    n = zeros_hbm.shape[0]
    e = dst.shape[0]
    bt = 64
    per_t = e // 16
    iters = per_t // bt
    tail = per_t - iters * bt
    big = 640
    last = n - 15 * big
    assert tail % 8 == 0 and last > 0 and big % 8 == 0 and last % 8 == 0

    def body(msg_hbm, dst_hbm, zer_hbm, out_hbm, rows_v, idx_v, rows_t, idx_t,
             accum_sh, sem_l, sem_m, sem_sc):
        cid = lax.axis_index("c")
        sid = lax.axis_index("s")
        ebase = sid * per_t
        nbase = sid * big

        def stripe_copy(src_at, dst_at):
            @pl.when(sid < 15)
            def _():
                pltpu.sync_copy(src_at(nbase, big), dst_at(nbase, big))

            @pl.when(sid == 15)
            def _():
                pltpu.sync_copy(src_at(15 * big, last), dst_at(15 * big, last))

        for p in range(2):
            s_val = cid * 2 + p
            stripe_copy(lambda o, w: zer_hbm.at[pl.ds(o, w)],
                        lambda o, w: accum_sh.at[pl.ds(o, w)])
            plsc.subcore_barrier()

            def loads(j, buf):
                off = ebase + j * bt
                pltpu.async_copy(dst_hbm.at[pl.ds(off, bt)],
                                 idx_v.at[buf], sem_l.at[buf])
                pltpu.async_copy(msg_hbm.at[pl.ds(s_val * e + off, bt)],
                                 rows_v.at[buf], sem_m.at[buf])

            loads(0, 0)

            @pl.when(iters > 1)
            def _():
                loads(1, 1)

            def step(j, c):
                bc = j % 4
                pltpu.make_async_copy(dst_hbm.at[pl.ds(ebase, bt)],
                                      idx_v.at[bc], sem_l.at[bc]).wait()
                pltpu.make_async_copy(msg_hbm.at[pl.ds(ebase, bt)],
                                      rows_v.at[bc], sem_m.at[bc]).wait()
                pltpu.async_copy(rows_v.at[bc], accum_sh.at[idx_v.at[bc]],
                                 sem_sc.at[bc], add=True)

                @pl.when(j + 2 < iters)
                def _():
                    bn = (j + 2) % 4

                    @pl.when(j >= 2)
                    def _():
                        pltpu.make_async_copy(
                            rows_v.at[bn], accum_sh.at[idx_v.at[bn]],
                            sem_sc.at[bn]).wait()

                    loads(j + 2, bn)

                return c

            lax.fori_loop(0, iters, step, 0)
            for jt in range(min(4, iters)):
                j = iters - min(4, iters) + jt
                bp = j % 4
                pltpu.make_async_copy(rows_v.at[bp],
                                      accum_sh.at[idx_v.at[bp]],
                                      sem_sc.at[bp]).wait()
            if tail:
                off = ebase + iters * bt
                pltpu.sync_copy(dst_hbm.at[pl.ds(off, tail)], idx_t)
                pltpu.sync_copy(msg_hbm.at[pl.ds(s_val * e + off, tail)],
                                rows_t)
                pltpu.sync_copy(rows_t, accum_sh.at[idx_t], add=True)
            plsc.subcore_barrier()
            stripe_copy(lambda o, w: accum_sh.at[pl.ds(o, w)],
                        lambda o, w: out_hbm.at[pl.ds(s_val * n + o, w)])
            plsc.subcore_barrier()

    fn = pl.kernel(
        body,
        out_type=jax.ShapeDtypeStruct((4 * n, 128), jnp.float32),
        mesh=plsc.VectorSubcoreMesh(core_axis_name="c", subcore_axis_name="s"),
        scratch_types=[
            pltpu.VMEM((4, bt, 128), jnp.float32),
            pltpu.VMEM((4, bt), jnp.int32),
            pltpu.VMEM((max(tail, 8), 128), jnp.float32),
            pltpu.VMEM((max(tail, 8),), jnp.int32),
            pltpu.VMEM_SHARED((n, 128), jnp.float32),
            pltpu.SemaphoreType.DMA((4,)),
            pltpu.SemaphoreType.DMA((4,)),
            pltpu.SemaphoreType.DMA((4,)),
        ],
    )
    return fn(msg_flat, dst, zeros_hbm)


def _silu(x):
    return x * jax.nn.sigmoid(x)


def _mm(a, b):
    return jnp.dot(a, b, preferred_element_type=jnp.float32)


def _rmlp_block(ef, w1, b1, w2, b2, w3, b3, w4):
    h = _silu(_mm(ef, w1) + b1)
    h = _silu(_mm(h, w2) + b2)
    h = _silu(_mm(h, w3) + b3)
    return _mm(h, w4)


def _dot00(a, b):
    return lax.dot_general(a, b, (((0,), (0,)), ((), ())),
                           preferred_element_type=jnp.float32)


def _rmlp_t(eft, w1, b1, w2, b2, w3, b3, w4):
    h = _silu(_dot00(w1, eft) + b1)
    h = _silu(_dot00(w2, h) + b2)
    h = _silu(_dot00(w3, h) + b3)
    return _dot00(w4, h)


def _edge1_body(ps_ref, pd_ref, sh_ref, hs_ref,
                aw1, ab1, aw2, ab2, aw3, ab3, aw4,
                bw1, bb1, bw2, bb2, bw3, bb3, bw4,
                msg_ref, r1_ref, y_ref):
    vec = pd_ref[...] - ps_ref[...] + sh_ref[...]
    vect = jnp.transpose(vec)
    len2 = jnp.sum(vect * vect, axis=0, keepdims=True) + 1e-18
    rt = jnp.sqrt(len2)
    ut = vect * (1.0 / rt)

    rc = jnp.minimum(jnp.maximum(rt, 1e-6), RMAX)
    ncol = (lax.broadcasted_iota(jnp.int32, (8, 1), 0) + 1).astype(jnp.float32)
    rb = jnp.sqrt(2.0 / RMAX) * jnp.sin((rc * (jnp.pi / RMAX)) * ncol) / rc
    uu = jnp.clip(rt / RMAX, 0.0, 1.0)
    u5 = uu * uu * uu * uu * uu
    env = 1.0 - 21.0 * u5 + 35.0 * u5 * uu - 15.0 * u5 * uu * uu
    env = jnp.where(rt < RMAX, env, 0.0)
    eft = rb * env

    r0t = _rmlp_t(eft, aw1[...], ab1[...], aw2[...], ab2[...],
                  aw3[...], ab3[...], aw4[...])
    r1t = _rmlp_t(eft, bw1[...], bb1[...], bw2[...], bb2[...],
                  bw3[...], bb3[...], bw4[...])
    r1_ref[...] = jnp.transpose(r1t)
    ones = jnp.ones_like(ut[0:1])
    yt = jnp.concatenate(
        [ones, S3 * ut[0:1], S3 * ut[1:2], S3 * ut[2:3]], axis=0)
    y = jnp.transpose(yt)
    y_ref[...] = y
    r0 = jnp.transpose(r0t)
    hs = hs_ref[...]
    for s in range(4):
        rs = r0[:, LMAP[s] * 128:(LMAP[s] + 1) * 128]
        if s == 0:
            msg_ref[s] = rs * hs
        else:
            msg_ref[s] = rs * (y[:, s:s + 1] * hs)


def _edge2_body(r1_ref, y_ref, h2s_ref, msg_ref):
    r1 = r1_ref[...]
    y = y_ref[...]
    h2s = h2s_ref[...]
    h0 = h2s[:, 0:128]
    for s in range(4):
        rs = r1[:, LMAP[s] * 128:(LMAP[s] + 1) * 128]
        msg_ref[s] = rs * (y[:, s:s + 1] * h0 + h2s[:, s * 128:(s + 1) * 128])


def _node0_body(na_ref, wemb, wup0, h_ref):
    h_ref[...] = _mm(_mm(na_ref[...], wemb[...]), wup0[...])


def _graph_accum(eg_ref, batch_col, en_node):
    iot = lax.broadcasted_iota(jnp.int32, (batch_col.shape[0], 16), 1)
    oh = (batch_col == iot).astype(jnp.float32)
    part = lax.dot_general(oh, en_node, (((0,), (0,)), ((), ())),
                           preferred_element_type=jnp.float32)
    i = pl.program_id(0)

    @pl.when(i == 0)
    def _():
        eg_ref[...] = jnp.zeros_like(eg_ref)

    eg_ref[...] += part


def _node1_body(agg_ref, na_ref, bat_ref,
                wout_r, pw_r, pv_r, wsc_r, wmix_r, wmixv_r, wemb_r, wup1_r,
                ae_r, ro0_r, h2_ref, scal1_ref, eg_ref):
    wout, pw, pv, wsc, wmix, wmixv, wemb, wup1, ae, ro0 = (
        wout_r[...], pw_r[...], pv_r[...], wsc_r[...], wmix_r[...],
        wmixv_r[...], wemb_r[...], wup1_r[...], ae_r[...], ro0_r[...])
    na = na_ref[...]
    nf = _mm(na, wemb)
    out0 = _mm(agg_ref[0] * (1.0 / AVG), wout[0])
    outv = [_mm(agg_ref[s] * (1.0 / AVG), wout[1]) for s in range(1, 4)]
    w0 = _mm(na, pw[0])
    w1 = _mm(na, pw[1])
    w2 = _mm(na, pw[2])
    vsq = outv[0] * outv[0] + outv[1] * outv[1] + outv[2] * outv[2]
    scal = w0 * out0 + w1 * out0 * out0 + w2 * vsq
    sc = _mm(na, wsc) * nf
    scal = _mm(scal, wmix) + sc
    pv0 = _mm(na, pv[0])
    pv1 = _mm(na, pv[1])
    coef = pv0 + pv1 * out0
    vout = [_mm(coef * v, wmixv) for v in outv]

    scal1_ref[...] = scal
    h2_ref[:, 0:128] = _mm(scal, wup1[0])
    for s in range(1, 4):
        h2_ref[:, s * 128:(s + 1) * 128] = _mm(vout[s - 1], wup1[1])

    e01 = _mm(na, ae) + _mm(scal, ro0)
    _graph_accum(eg_ref, bat_ref[...], e01)


def _node2_body(agg_ref, na_ref, bat_ref, scal1_ref,
                wout_r, pw_r, wsc_r, wmix_r, row1_r, rob1, row2_r,
                eg_ref):
    wout, pw, wsc, wmix, row1, row2 = (
        wout_r[...], pw_r[...], wsc_r[...], wmix_r[...], row1_r[...],
        row2_r[...])
    na = na_ref[...]
    out0 = _mm(agg_ref[0] * (1.0 / AVG), wout[0])
    outv = [_mm(agg_ref[s] * (1.0 / AVG), wout[1]) for s in range(1, 4)]
    w0 = _mm(na, pw[0])
    w1 = _mm(na, pw[1])
    w2 = _mm(na, pw[2])
    vsq = outv[0] * outv[0] + outv[1] * outv[1] + outv[2] * outv[2]
    scal2 = w0 * out0 + w1 * out0 * out0 + w2 * vsq
    sc2 = _mm(na, wsc) * scal1_ref[...]
    scal2 = _mm(scal2, wmix) + sc2
    hr = _silu(_mm(scal2, row1) + rob1[...])
    en2 = _mm(hr, row2)
    _graph_accum(eg_ref, bat_ref[...], en2)


def _full(shape):
    nd = len(shape)
    return pl.BlockSpec(shape, lambda i: (0,) * nd)


def _run_edge1(posg, shifts16, h_src, lw, e, b1, interpret=False):
    g = e // b1
    specs = [
        pl.BlockSpec((b1, 16), lambda i: (i, 0)),
        pl.BlockSpec((b1, 16), lambda i: (i + g, 0)),
        pl.BlockSpec((b1, 16), lambda i: (i, 0)),
        pl.BlockSpec((b1, 128), lambda i: (i, 0)),
        _full((8, 64)), _full((64, 1)), _full((64, 64)), _full((64, 1)),
        _full((64, 64)), _full((64, 1)), _full((64, 256)),
        _full((8, 64)), _full((64, 1)), _full((64, 64)), _full((64, 1)),
        _full((64, 64)), _full((64, 1)), _full((64, 256)),
    ]
    out_specs = [
        pl.BlockSpec((4, b1, 128), lambda i: (0, i, 0)),
        pl.BlockSpec((b1, 256), lambda i: (i, 0)),
        pl.BlockSpec((b1, 4), lambda i: (i, 0)),
    ]
    out_shape = [
        jax.ShapeDtypeStruct((4, e, 128), jnp.float32),
        jax.ShapeDtypeStruct((e, 256), jnp.float32),
        jax.ShapeDtypeStruct((e, 4), jnp.float32),
    ]
    return pl.pallas_call(
        _edge1_body, grid=(g,), in_specs=specs, out_specs=out_specs,
        out_shape=out_shape, interpret=interpret,
    )(posg, posg, shifts16, h_src, *lw)


def _run_edge2(r1, y, h2s, e, b1, interpret=False):
    g = e // b1
    specs = [
        pl.BlockSpec((b1, 256), lambda i: (i, 0)),
        pl.BlockSpec((b1, 4), lambda i: (i, 0)),
        pl.BlockSpec((b1, 512), lambda i: (i, 0)),
    ]
    return pl.pallas_call(
        _edge2_body, grid=(g,), in_specs=specs,
        out_specs=pl.BlockSpec((4, b1, 128), lambda i: (0, i, 0)),
        out_shape=jax.ShapeDtypeStruct((4, e, 128), jnp.float32),
        interpret=interpret,
    )(r1, y, h2s)


def _run_node0(na, wemb, wup0, n, bn, interpret=False):
    return pl.pallas_call(
        _node0_body, grid=(n // bn,),
        in_specs=[pl.BlockSpec((bn, 10), lambda i: (i, 0)),
                  _full((10, 128)), _full((128, 128))],
        out_specs=pl.BlockSpec((bn, 128), lambda i: (i, 0)),
        out_shape=jax.ShapeDtypeStruct((n, 128), jnp.float32),
        interpret=interpret,
    )(na, wemb, wup0)


def _run_node1(agg0, na, bat2, wts, n, bn, interpret=False):
    g = n // bn
    specs = [
        pl.BlockSpec((4, bn, 128), lambda i: (0, i, 0)),
        pl.BlockSpec((bn, 10), lambda i: (i, 0)),
        pl.BlockSpec((bn, 1), lambda i: (i, 0)),
        _full((2, 128, 128)), _full((3, 10, 128)), _full((2, 10, 128)),
        _full((10, 128)), _full((128, 128)), _full((128, 128)),
        _full((10, 128)), _full((2, 128, 128)), _full((10, 1)),
        _full((128, 1)),
    ]
    out_specs = [
        pl.BlockSpec((bn, 512), lambda i: (i, 0)),
        pl.BlockSpec((bn, 128), lambda i: (i, 0)),
        pl.BlockSpec((16, 1), lambda i: (0, 0)),
    ]
    out_shape = [
        jax.ShapeDtypeStruct((n, 512), jnp.float32),
        jax.ShapeDtypeStruct((n, 128), jnp.float32),
        jax.ShapeDtypeStruct((16, 1), jnp.float32),
    ]
    return pl.pallas_call(
        _node1_body, grid=(g,), in_specs=specs, out_specs=out_specs,
        out_shape=out_shape, interpret=interpret,
    )(agg0, na, bat2, *wts)


def _run_node2(agg2, na, bat2, scal1, wts, n, bn, interpret=False):
    g = n // bn
    specs = [
        pl.BlockSpec((4, bn, 128), lambda i: (0, i, 0)),
        pl.BlockSpec((bn, 10), lambda i: (i, 0)),
        pl.BlockSpec((bn, 1), lambda i: (i, 0)),
        pl.BlockSpec((bn, 128), lambda i: (i, 0)),
        _full((2, 128, 128)), _full((3, 10, 128)), _full((10, 128)),
        _full((128, 128)), _full((128, 16)), _full((1, 16)), _full((16, 1)),
    ]
    return pl.pallas_call(
        _node2_body, grid=(g,), in_specs=specs,
        out_specs=pl.BlockSpec((16, 1), lambda i: (0, 0)),
        out_shape=jax.ShapeDtypeStruct((16, 1), jnp.float32),
        interpret=interpret,
    )(agg2, na, bat2, scal1, *wts)


def kernel(positions, node_attrs, shifts, params, edge_index, batch, ptr):
    n = positions.shape[0]
    e = edge_index.shape[1]
    ng = ptr.shape[0] - 1
    b1 = 1000
    bn = 1000

    p0 = params["layer0"]
    p1 = params["layer1"]
    l0w = (p0["rW1"], p0["rb1"].reshape(64, 1), p0["rW2"], p0["rb2"].reshape(64, 1),
           p0["rW3"], p0["rb3"].reshape(64, 1), p0["rW4"],
           p1["rW1"], p1["rb1"].reshape(64, 1), p1["rW2"], p1["rb2"].reshape(64, 1),
           p1["rW3"], p1["rb3"].reshape(64, 1), p1["rW4"])

    pos16 = jnp.pad(positions, ((0, 0), (0, 13)))
    sh16 = jnp.pad(shifts, ((0, 0), (0, 13)))
    eidx = edge_index.reshape(2 * e)
    src = edge_index[0]
    dst = edge_index[1]
    bat2 = batch.reshape(n, 1)
    zeros_n = jnp.zeros((n, 128), jnp.float32)

    h = _run_node0(node_attrs, params["W_embed"], p0["W_up"], n, bn)

    posg = _sc_gather_rows(pos16, eidx, 128, tc_tiling=False)
    h_src = _sc_gather_rows(h, src, 128)

    msg0, r1e, y = _run_edge1(posg, sh16, h_src, l0w, e, b1)
    agg0 = _sc_scatter4(msg0.reshape(4 * e, 128), dst, zeros_n).reshape(4, n, 128)

    wts1 = (p0["W_out"], p0["pw"], p0["pv"], p0["W_sc"], p0["W_mix"],
            p0["W_mixv"], params["W_embed"], p1["W_up"],
            params["atomic_energies"].reshape(10, 1),
            params["readout0"].reshape(128, 1))
    h2, scal1, e01g = _run_node1(agg0, node_attrs, bat2, wts1, n, bn)

    h2s = _sc_gather_rows(h2, src, 56)
    msg2 = _run_edge2(r1e, y, h2s, e, b1)
    agg2 = _sc_scatter4(msg2.reshape(4 * e, 128), dst, zeros_n).reshape(4, n, 128)

    wts2 = (p1["W_out"], p1["pw"], p1["W_sc"], p1["W_mix"],
            params["ro1_W1"], params["ro1_b1"].reshape(1, 16),
            params["ro1_W2"])
    e2g = _run_node2(agg2, node_attrs, bat2, scal1, wts2, n, bn)

    return (e01g + e2g).reshape(ng)


# speedup vs baseline: 1.2280x; 1.0584x over previous
import jax
import jax.numpy as jnp
from jax import lax
from jax.experimental import pallas as pl
from jax.experimental.pallas import tpu as pltpu
from jax.experimental.pallas import tpu_sc as plsc

RMAX = 5.0
AVG = 16.0
LMAP = (0, 1, 1, 1)
S3 = 1.7320508075688772


def _sc_gather_rows(table, idx, bt, tc_tiling=True):
    v, d = table.shape
    b = idx.shape[0]
    nw = 32
    per_w = b // nw
    iters = per_w // bt
    tail = per_w - iters * bt
    assert per_w * nw == b and bt % 8 == 0 and bt <= 128 and tail % 8 == 0

    def body(table_hbm, idx_hbm, out_hbm, idx_v, rows_v, sem_i, sem_g, sem_s):
        wid = lax.axis_index("s") * 2 + lax.axis_index("c")
        base = wid * per_w

        def load_idx(j, buf):
            pltpu.async_copy(idx_hbm.at[pl.ds(base + j * bt, bt)],
                             idx_v.at[buf], sem_i.at[buf])

        load_idx(0, 0)

        @pl.when(iters > 1)
        def _():
            load_idx(1, 1)

        def step(j, c):
            bc = j % 4
            pltpu.make_async_copy(idx_hbm.at[pl.ds(base, bt)],
                                  idx_v.at[bc], sem_i.at[bc]).wait()

            @pl.when(j >= 4)
            def _():
                pltpu.make_async_copy(rows_v.at[bc],
                                      out_hbm.at[pl.ds(base, bt)],
                                      sem_s.at[bc]).wait()

            pltpu.async_copy(table_hbm.at[idx_v.at[bc]], rows_v.at[bc],
                             sem_g.at[bc])

            @pl.when(j >= 2)
            def _():
                bp = (j - 2) % 4
                pltpu.make_async_copy(table_hbm.at[idx_v.at[bp]],
                                      rows_v.at[bp], sem_g.at[bp]).wait()
                pltpu.async_copy(rows_v.at[bp],
                                 out_hbm.at[pl.ds(base + (j - 2) * bt, bt)],
                                 sem_s.at[bp])

            @pl.when(j + 2 < iters)
            def _():
                load_idx(j + 2, (j + 2) % 4)

            return c

        lax.fori_loop(0, iters, step, 0)
        for jt in range(2):
            j = iters - 2 + jt
            if j < 0:
                continue
            bp = j % 4
            pltpu.make_async_copy(table_hbm.at[idx_v.at[bp]],
                                  rows_v.at[bp], sem_g.at[bp]).wait()
            pltpu.async_copy(rows_v.at[bp],
                             out_hbm.at[pl.ds(base + j * bt, bt)],
                             sem_s.at[bp])
        for jt in range(min(4, iters)):
            j = iters - min(4, iters) + jt
            bp = j % 4
            pltpu.make_async_copy(rows_v.at[bp],
                                  out_hbm.at[pl.ds(base, bt)],
                                  sem_s.at[bp]).wait()
        if tail:
            off = base + iters * bt
            pltpu.sync_copy(idx_hbm.at[pl.ds(off, tail)],
                            idx_v.at[0, pl.ds(0, tail)])
            pltpu.async_copy(table_hbm.at[idx_v.at[0, pl.ds(0, tail)]],
                             rows_v.at[0, pl.ds(0, tail)], sem_g.at[0]).wait()
            pltpu.sync_copy(rows_v.at[0, pl.ds(0, tail)],
                            out_hbm.at[pl.ds(off, tail)])

    fn = pl.kernel(
        body,
        out_type=jax.ShapeDtypeStruct((b, d), jnp.float32),
        mesh=plsc.VectorSubcoreMesh(core_axis_name="c", subcore_axis_name="s"),
        scratch_types=[
            pltpu.VMEM((4, bt), jnp.int32),
            pltpu.VMEM((4, bt, d), jnp.float32),
            pltpu.SemaphoreType.DMA((4,)),
            pltpu.SemaphoreType.DMA((4,)),
            pltpu.SemaphoreType.DMA((4,)),
        ],
        compiler_params=pltpu.CompilerParams(use_tc_tiling_on_sc=tc_tiling),
    )
    return fn(table, idx)


def _sc_scatter4(msg_flat, dst, zeros_hbm):
    n = zeros_hbm.shape[0]
    e = dst.shape[0]
    bt = 64
    per_t = e // 16
    iters = per_t // bt
    tail = per_t - iters * bt
    big = 640
    last = n - 15 * big
    assert tail % 8 == 0 and last > 0 and big % 8 == 0 and last % 8 == 0

    def body(msg_hbm, dst_hbm, zer_hbm, out_hbm, rows_v, idx_v, rows_t, idx_t,
             accum_sh, sem_l, sem_m, sem_sc):
        cid = lax.axis_index("c")
        sid = lax.axis_index("s")
        ebase = sid * per_t
        nbase = sid * big

        def stripe_copy(src_at, dst_at):
            @pl.when(sid < 15)
            def _():
                pltpu.sync_copy(src_at(nbase, big), dst_at(nbase, big))

            @pl.when(sid == 15)
            def _():
                pltpu.sync_copy(src_at(15 * big, last), dst_at(15 * big, last))

        for p in range(2):
            s_val = cid * 2 + p
            stripe_copy(lambda o, w: zer_hbm.at[pl.ds(o, w)],
                        lambda o, w: accum_sh.at[pl.ds(o, w)])
            plsc.subcore_barrier()

            def loads(j, buf):
                off = ebase + j * bt
                pltpu.async_copy(dst_hbm.at[pl.ds(off, bt)],
                                 idx_v.at[buf], sem_l.at[buf])
                pltpu.async_copy(msg_hbm.at[pl.ds(s_val * e + off, bt)],
                                 rows_v.at[buf], sem_m.at[buf])

            loads(0, 0)

            @pl.when(iters > 1)
            def _():
                loads(1, 1)

            def step(j, c):
                bc = j % 4
                pltpu.make_async_copy(dst_hbm.at[pl.ds(ebase, bt)],
                                      idx_v.at[bc], sem_l.at[bc]).wait()
                pltpu.make_async_copy(msg_hbm.at[pl.ds(ebase, bt)],
                                      rows_v.at[bc], sem_m.at[bc]).wait()
                pltpu.async_copy(rows_v.at[bc], accum_sh.at[idx_v.at[bc]],
                                 sem_sc.at[bc], add=True)

                @pl.when(j + 2 < iters)
                def _():
                    bn = (j + 2) % 4

                    @pl.when(j >= 2)
                    def _():
                        pltpu.make_async_copy(
                            rows_v.at[bn], accum_sh.at[idx_v.at[bn]],
                            sem_sc.at[bn]).wait()

                    loads(j + 2, bn)

                return c

            lax.fori_loop(0, iters, step, 0)
            for jt in range(min(4, iters)):
                j = iters - min(4, iters) + jt
                bp = j % 4
                pltpu.make_async_copy(rows_v.at[bp],
                                      accum_sh.at[idx_v.at[bp]],
                                      sem_sc.at[bp]).wait()
            if tail:
                off = ebase + iters * bt
                pltpu.sync_copy(dst_hbm.at[pl.ds(off, tail)], idx_t)
                pltpu.sync_copy(msg_hbm.at[pl.ds(s_val * e + off, tail)],
                                rows_t)
                pltpu.sync_copy(rows_t, accum_sh.at[idx_t], add=True)
            plsc.subcore_barrier()
            stripe_copy(lambda o, w: accum_sh.at[pl.ds(o, w)],
                        lambda o, w: out_hbm.at[pl.ds(s_val * n + o, w)])
            plsc.subcore_barrier()

    fn = pl.kernel(
        body,
        out_type=jax.ShapeDtypeStruct((4 * n, 128), jnp.float32),
        mesh=plsc.VectorSubcoreMesh(core_axis_name="c", subcore_axis_name="s"),
        scratch_types=[
            pltpu.VMEM((4, bt, 128), jnp.float32),
            pltpu.VMEM((4, bt), jnp.int32),
            pltpu.VMEM((max(tail, 8), 128), jnp.float32),
            pltpu.VMEM((max(tail, 8),), jnp.int32),
            pltpu.VMEM_SHARED((n, 128), jnp.float32),
            pltpu.SemaphoreType.DMA((4,)),
            pltpu.SemaphoreType.DMA((4,)),
            pltpu.SemaphoreType.DMA((4,)),
        ],
    )
    return fn(msg_flat, dst, zeros_hbm)


def _silu(x):
    return x * jax.nn.sigmoid(x)


def _mm(a, b):
    return jnp.dot(a, b, preferred_element_type=jnp.float32)


def _rmlp_block(ef, w1, b1, w2, b2, w3, b3, w4):
    h = _silu(_mm(ef, w1) + b1)
    h = _silu(_mm(h, w2) + b2)
    h = _silu(_mm(h, w3) + b3)
    return _mm(h, w4)


def _dot00(a, b):
    return lax.dot_general(a, b, (((0,), (0,)), ((), ())),
                           preferred_element_type=jnp.float32)


def _rmlp_t(eft, w1, b1, w2, b2, w3, b3, w4):
    h = _silu(_dot00(w1, eft) + b1)
    h = _silu(_dot00(w2, h) + b2)
    h = _silu(_dot00(w3, h) + b3)
    return _dot00(w4, h)


def _edge1_body(ps_ref, pd_ref, sh_ref, hs_ref,
                aw1, ab1, aw2, ab2, aw3, ab3, aw4,
                bw1, bb1, bw2, bb2, bw3, bb3, bw4,
                msg_ref, r1_ref, y_ref):
    vec = pd_ref[...] - ps_ref[...] + sh_ref[...]
    vect = jnp.transpose(vec)
    len2 = jnp.sum(vect * vect, axis=0, keepdims=True) + 1e-18
    rt = jnp.sqrt(len2)
    ut = vect * (1.0 / rt)

    rc = jnp.minimum(jnp.maximum(rt, 1e-6), RMAX)
    ncol = (lax.broadcasted_iota(jnp.int32, (8, 1), 0) + 1).astype(jnp.float32)
    rb = jnp.sqrt(2.0 / RMAX) * jnp.sin((rc * (jnp.pi / RMAX)) * ncol) / rc
    uu = jnp.clip(rt / RMAX, 0.0, 1.0)
    u5 = uu * uu * uu * uu * uu
    env = 1.0 - 21.0 * u5 + 35.0 * u5 * uu - 15.0 * u5 * uu * uu
    env = jnp.where(rt < RMAX, env, 0.0)
    eft = rb * env

    r0t = _rmlp_t(eft, aw1[...], ab1[...], aw2[...], ab2[...],
                  aw3[...], ab3[...], aw4[...])
    r1t = _rmlp_t(eft, bw1[...], bb1[...], bw2[...], bb2[...],
                  bw3[...], bb3[...], bw4[...])
    r1_ref[...] = jnp.transpose(r1t)
    ones = jnp.ones_like(ut[0:1])
    yt = jnp.concatenate(
        [ones, S3 * ut[0:1], S3 * ut[1:2], S3 * ut[2:3]], axis=0)
    y = jnp.transpose(yt)
    y_ref[...] = y
    r0 = jnp.transpose(r0t)
    hs = hs_ref[...]
    for s in range(4):
        rs = r0[:, LMAP[s] * 128:(LMAP[s] + 1) * 128]
        if s == 0:
            msg_ref[s] = rs * hs
        else:
            msg_ref[s] = rs * (y[:, s:s + 1] * hs)


def _edge2_body(r1_ref, y_ref, h2s_ref, msg_ref):
    r1 = r1_ref[...]
    y = y_ref[...]
    h2s = h2s_ref[...]
    h0 = h2s[:, 0:128]
    for s in range(4):
        rs = r1[:, LMAP[s] * 128:(LMAP[s] + 1) * 128]
        msg_ref[s] = rs * (y[:, s:s + 1] * h0 + h2s[:, s * 128:(s + 1) * 128])


def _node0_body(na_ref, wemb, wup0, h_ref):
    h_ref[...] = _mm(_mm(na_ref[...], wemb[...]), wup0[...])


def _graph_accum(eg_ref, batch_col, en_node):
    iot = lax.broadcasted_iota(jnp.int32, (batch_col.shape[0], 16), 1)
    oh = (batch_col == iot).astype(jnp.float32)
    part = lax.dot_general(oh, en_node, (((0,), (0,)), ((), ())),
                           preferred_element_type=jnp.float32)
    i = pl.program_id(0)

    @pl.when(i == 0)
    def _():
        eg_ref[...] = jnp.zeros_like(eg_ref)

    eg_ref[...] += part


def _node1_body(agg_ref, agb_ref, na_ref, bat_ref,
                wout_r, pw_r, pv_r, wsc_r, wmix_r, wmixv_r, wemb_r, wup1_r,
                ae_r, ro0_r, h2_ref, scal1_ref, eg_ref):
    wout, pw, pv, wsc, wmix, wmixv, wemb, wup1, ae, ro0 = (
        wout_r[...], pw_r[...], pv_r[...], wsc_r[...], wmix_r[...],
        wmixv_r[...], wemb_r[...], wup1_r[...], ae_r[...], ro0_r[...])
    na = na_ref[...]
    nf = _mm(na, wemb)
    agg = [(agg_ref[s] + agb_ref[s]) * (1.0 / AVG) for s in range(4)]
    out0 = _mm(agg[0], wout[0])
    outv = [_mm(agg[s], wout[1]) for s in range(1, 4)]
    w0 = _mm(na, pw[0])
    w1 = _mm(na, pw[1])
    w2 = _mm(na, pw[2])
    vsq = outv[0] * outv[0] + outv[1] * outv[1] + outv[2] * outv[2]
    scal = w0 * out0 + w1 * out0 * out0 + w2 * vsq
    sc = _mm(na, wsc) * nf
    scal = _mm(scal, wmix) + sc
    pv0 = _mm(na, pv[0])
    pv1 = _mm(na, pv[1])
    coef = pv0 + pv1 * out0
    vout = [_mm(coef * v, wmixv) for v in outv]

    scal1_ref[...] = scal
    h2_ref[:, 0:128] = _mm(scal, wup1[0])
    for s in range(1, 4):
        h2_ref[:, s * 128:(s + 1) * 128] = _mm(vout[s - 1], wup1[1])

    e01 = _mm(na, ae) + _mm(scal, ro0)
    _graph_accum(eg_ref, bat_ref[...], e01)


def _node2_body(agg_ref, agb_ref, na_ref, bat_ref, scal1_ref,
                wout_r, pw_r, wsc_r, wmix_r, row1_r, rob1, row2_r,
                eg_ref):
    wout, pw, wsc, wmix, row1, row2 = (
        wout_r[...], pw_r[...], wsc_r[...], wmix_r[...], row1_r[...],
        row2_r[...])
    na = na_ref[...]
    agg = [(agg_ref[s] + agb_ref[s]) * (1.0 / AVG) for s in range(4)]
    out0 = _mm(agg[0], wout[0])
    outv = [_mm(agg[s], wout[1]) for s in range(1, 4)]
    w0 = _mm(na, pw[0])
    w1 = _mm(na, pw[1])
    w2 = _mm(na, pw[2])
    vsq = outv[0] * outv[0] + outv[1] * outv[1] + outv[2] * outv[2]
    scal2 = w0 * out0 + w1 * out0 * out0 + w2 * vsq
    sc2 = _mm(na, wsc) * scal1_ref[...]
    scal2 = _mm(scal2, wmix) + sc2
    hr = _silu(_mm(scal2, row1) + rob1[...])
    en2 = _mm(hr, row2)
    _graph_accum(eg_ref, bat_ref[...], en2)


def _full(shape):
    nd = len(shape)
    return pl.BlockSpec(shape, lambda i: (0,) * nd)


def _run_edge1(posg, shifts16, h_src, lw, e, b1, i0, eh, interpret=False):
    g = e // b1
    gh = eh // b1
    specs = [
        pl.BlockSpec((b1, 16), lambda i: (i + i0, 0)),
        pl.BlockSpec((b1, 16), lambda i: (i + i0 + g, 0)),
        pl.BlockSpec((b1, 16), lambda i: (i + i0, 0)),
        pl.BlockSpec((b1, 128), lambda i: (i + i0, 0)),
        _full((8, 64)), _full((64, 1)), _full((64, 64)), _full((64, 1)),
        _full((64, 64)), _full((64, 1)), _full((64, 256)),
        _full((8, 64)), _full((64, 1)), _full((64, 64)), _full((64, 1)),
        _full((64, 64)), _full((64, 1)), _full((64, 256)),
    ]
    out_specs = [
        pl.BlockSpec((4, b1, 128), lambda i: (0, i, 0)),
        pl.BlockSpec((b1, 256), lambda i: (i, 0)),
        pl.BlockSpec((b1, 4), lambda i: (i, 0)),
    ]
    out_shape = [
        jax.ShapeDtypeStruct((4, eh, 128), jnp.float32),
        jax.ShapeDtypeStruct((eh, 256), jnp.float32),
        jax.ShapeDtypeStruct((eh, 4), jnp.float32),
    ]
    return pl.pallas_call(
        _edge1_body, grid=(gh,), in_specs=specs, out_specs=out_specs,
        out_shape=out_shape, interpret=interpret,
    )(posg, posg, shifts16, h_src, *lw)


def _run_edge2(r1, y, h2s, e, b1, i0, eh, interpret=False):
    gh = eh // b1
    specs = [
        pl.BlockSpec((b1, 256), lambda i: (i, 0)),
        pl.BlockSpec((b1, 4), lambda i: (i, 0)),
        pl.BlockSpec((b1, 512), lambda i: (i + i0, 0)),
    ]
    return pl.pallas_call(
        _edge2_body, grid=(gh,), in_specs=specs,
        out_specs=pl.BlockSpec((4, b1, 128), lambda i: (0, i, 0)),
        out_shape=jax.ShapeDtypeStruct((4, eh, 128), jnp.float32),
        interpret=interpret,
    )(r1, y, h2s)


def _run_node0(na, wemb, wup0, n, bn, interpret=False):
    return pl.pallas_call(
        _node0_body, grid=(n // bn,),
        in_specs=[pl.BlockSpec((bn, 10), lambda i: (i, 0)),
                  _full((10, 128)), _full((128, 128))],
        out_specs=pl.BlockSpec((bn, 128), lambda i: (i, 0)),
        out_shape=jax.ShapeDtypeStruct((n, 128), jnp.float32),
        interpret=interpret,
    )(na, wemb, wup0)


def _run_node1(agg0, agg0b, na, bat2, wts, n, bn, interpret=False):
    g = n // bn
    specs = [
        pl.BlockSpec((4, bn, 128), lambda i: (0, i, 0)),
        pl.BlockSpec((4, bn, 128), lambda i: (0, i, 0)),
        pl.BlockSpec((bn, 10), lambda i: (i, 0)),
        pl.BlockSpec((bn, 1), lambda i: (i, 0)),
        _full((2, 128, 128)), _full((3, 10, 128)), _full((2, 10, 128)),
        _full((10, 128)), _full((128, 128)), _full((128, 128)),
        _full((10, 128)), _full((2, 128, 128)), _full((10, 1)),
        _full((128, 1)),
    ]
    out_specs = [
        pl.BlockSpec((bn, 512), lambda i: (i, 0)),
        pl.BlockSpec((bn, 128), lambda i: (i, 0)),
        pl.BlockSpec((16, 1), lambda i: (0, 0)),
    ]
    out_shape = [
        jax.ShapeDtypeStruct((n, 512), jnp.float32),
        jax.ShapeDtypeStruct((n, 128), jnp.float32),
        jax.ShapeDtypeStruct((16, 1), jnp.float32),
    ]
    return pl.pallas_call(
        _node1_body, grid=(g,), in_specs=specs, out_specs=out_specs,
        out_shape=out_shape, interpret=interpret,
    )(agg0, agg0b, na, bat2, *wts)


def _run_node2(agg2, agg2b, na, bat2, scal1, wts, n, bn, interpret=False):
    g = n // bn
    specs = [
        pl.BlockSpec((4, bn, 128), lambda i: (0, i, 0)),
        pl.BlockSpec((4, bn, 128), lambda i: (0, i, 0)),
        pl.BlockSpec((bn, 10), lambda i: (i, 0)),
        pl.BlockSpec((bn, 1), lambda i: (i, 0)),
        pl.BlockSpec((bn, 128), lambda i: (i, 0)),
        _full((2, 128, 128)), _full((3, 10, 128)), _full((10, 128)),
        _full((128, 128)), _full((128, 16)), _full((1, 16)), _full((16, 1)),
    ]
    return pl.pallas_call(
        _node2_body, grid=(g,), in_specs=specs,
        out_specs=pl.BlockSpec((16, 1), lambda i: (0, 0)),
        out_shape=jax.ShapeDtypeStruct((16, 1), jnp.float32),
        interpret=interpret,
    )(agg2, agg2b, na, bat2, scal1, *wts)


def kernel(positions, node_attrs, shifts, params, edge_index, batch, ptr):
    n = positions.shape[0]
    e = edge_index.shape[1]
    ng = ptr.shape[0] - 1
    b1 = 1000
    bn = 1000

    p0 = params["layer0"]
    p1 = params["layer1"]
    l0w = (p0["rW1"], p0["rb1"].reshape(64, 1), p0["rW2"], p0["rb2"].reshape(64, 1),
           p0["rW3"], p0["rb3"].reshape(64, 1), p0["rW4"],
           p1["rW1"], p1["rb1"].reshape(64, 1), p1["rW2"], p1["rb2"].reshape(64, 1),
           p1["rW3"], p1["rb3"].reshape(64, 1), p1["rW4"])

    pos16 = jnp.pad(positions, ((0, 0), (0, 13)))
    sh16 = jnp.pad(shifts, ((0, 0), (0, 13)))
    eidx = edge_index.reshape(2 * e)
    src = edge_index[0]
    dst = edge_index[1]
    bat2 = batch.reshape(n, 1)
    zeros_n = jnp.zeros((n, 128), jnp.float32)

    h = _run_node0(node_attrs, params["W_embed"], p0["W_up"], n, bn)

    posg = _sc_gather_rows(pos16, eidx, 128, tc_tiling=False)
    h_src = _sc_gather_rows(h, src, 128)

    eh = e // 2
    gh = eh // b1
    dst_a = dst[:eh]
    dst_b = dst[eh:]

    msg0a, r1a, ya = _run_edge1(posg, sh16, h_src, l0w, e, b1, 0, eh)
    agg0a = _sc_scatter4(msg0a.reshape(4 * eh, 128), dst_a, zeros_n)
    msg0b, r1b, yb = _run_edge1(posg, sh16, h_src, l0w, e, b1, gh, eh)
    agg0b = _sc_scatter4(msg0b.reshape(4 * eh, 128), dst_b, zeros_n)

    wts1 = (p0["W_out"], p0["pw"], p0["pv"], p0["W_sc"], p0["W_mix"],
            p0["W_mixv"], params["W_embed"], p1["W_up"],
            params["atomic_energies"].reshape(10, 1),
            params["readout0"].reshape(128, 1))
    h2, scal1, e01g = _run_node1(agg0a.reshape(4, n, 128),
                                 agg0b.reshape(4, n, 128),
                                 node_attrs, bat2, wts1, n, bn)

    h2s = _sc_gather_rows(h2, src, 56)
    msg2a = _run_edge2(r1a, ya, h2s, e, b1, 0, eh)
    agg2a = _sc_scatter4(msg2a.reshape(4 * eh, 128), dst_a, zeros_n)
    msg2b = _run_edge2(r1b, yb, h2s, e, b1, gh, eh)
    agg2b = _sc_scatter4(msg2b.reshape(4 * eh, 128), dst_b, zeros_n)

    wts2 = (p1["W_out"], p1["pw"], p1["W_sc"], p1["W_mix"],
            params["ro1_W1"], params["ro1_b1"].reshape(1, 16),
            params["ro1_W2"])
    e2g = _run_node2(agg2a.reshape(4, n, 128), agg2b.reshape(4, n, 128),
                     node_attrs, bat2, scal1, wts2, n, bn)

    return (e01g + e2g).reshape(ng)
